# baseline (speedup 1.0000x reference)
import functools

import jax
import jax.numpy as jnp
from jax import lax
from jax.experimental import pallas as pl
from jax.experimental.pallas import tpu as pltpu
from jax.experimental.pallas import tpu_sc as plsc

BATCH = 16384
D = 64
H = 128
GRP = 16
NUM_CORES = 2
NUM_SUBCORES = 16
NUM_WORKERS = NUM_CORES * NUM_SUBCORES
B_PER_W = BATCH // NUM_WORKERS
CHUNK = 64
N_CHUNKS = B_PER_W // CHUNK
K = 16


def _gather_sc(tbf, idx16):
    mesh = plsc.VectorSubcoreMesh(core_axis_name="c", subcore_axis_name="s")

    @functools.partial(
        pl.kernel,
        mesh=mesh,
        out_type=jax.ShapeDtypeStruct((BATCH * GRP, D), jnp.bfloat16),
        scratch_types=[
            pltpu.VMEM((B_PER_W,), jnp.int32),
            pltpu.VMEM((CHUNK * GRP, D), jnp.bfloat16),
            pltpu.SemaphoreType.DMA,
        ],
    )
    def k(tbf_hbm, idx_hbm, out_hbm, idx_v, rows_v, sem):
        wid = lax.axis_index("s") * NUM_CORES + lax.axis_index("c")
        base = wid * B_PER_W
        pltpu.sync_copy(idx_hbm.at[pl.ds(base, B_PER_W)], idx_v)

        for ch in range(N_CHUNKS):
            c0 = ch * CHUNK

            @pl.loop(0, CHUNK, step=K)
            def _(r0):
                vec = idx_v[pl.ds(c0 + r0, K)]
                for j in range(K):
                    g = vec[j]
                    pltpu.make_async_copy(
                        tbf_hbm.at[pl.ds(g * GRP, GRP)],
                        rows_v.at[pl.ds((r0 + j) * GRP, GRP)], sem).start()

            @pl.loop(0, CHUNK, step=K)
            def _(r0):
                for j in range(K):
                    pltpu.make_async_copy(
                        tbf_hbm.at[pl.ds(0, GRP)],
                        rows_v.at[pl.ds((r0 + j) * GRP, GRP)], sem).wait()

            pltpu.sync_copy(
                rows_v,
                out_hbm.at[pl.ds((base + c0) * GRP, CHUNK * GRP)])

    return k(tbf, idx16)


def _mlp_body(x_ref, oh_ref, w1_ref, b1_ref, w2_ref, b2_ref, o_ref):
    quads = x_ref[...].astype(jnp.float32)
    oh = oh_ref[...]
    x = jnp.sum(quads * oh[:, :, None], axis=1)
    h = jnp.dot(x, w1_ref[...], preferred_element_type=jnp.float32) + b1_ref[...]
    h = jnp.maximum(h, 0.0)
    y = jnp.dot(h, w2_ref[...], preferred_element_type=jnp.float32) + b2_ref[...]
    n = jnp.sqrt(jnp.sum(y * y, axis=1, keepdims=True))
    o_ref[...] = y / jnp.maximum(n, 1e-12)


def _mlp(groups, onehot, W1, b1, W2, b2):
    blk = 1024
    return pl.pallas_call(
        _mlp_body,
        grid=(BATCH // blk,),
        in_specs=[
            pl.BlockSpec((blk, GRP, D), lambda i: (i, 0, 0)),
            pl.BlockSpec((blk, GRP), lambda i: (i, 0)),
            pl.BlockSpec((D, H), lambda i: (0, 0)),
            pl.BlockSpec((1, H), lambda i: (0, 0)),
            pl.BlockSpec((H, D), lambda i: (0, 0)),
            pl.BlockSpec((1, D), lambda i: (0, 0)),
        ],
        out_specs=pl.BlockSpec((blk, D), lambda i: (i, 0)),
        out_shape=jax.ShapeDtypeStruct((BATCH, D), jnp.float32),
    )(groups, onehot, W1, b1.reshape(1, H), W2, b2.reshape(1, D))


def kernel(user_ids, table, W1, b1, W2, b2):
    ids = user_ids.astype(jnp.int32)
    tbf = table.astype(jnp.bfloat16)
    idx16 = ids // GRP
    onehot = (ids[:, None] % GRP ==
              jnp.arange(GRP, dtype=jnp.int32)[None, :]).astype(jnp.float32)
    flat = _gather_sc(tbf, idx16)
    groups = flat.reshape(BATCH, GRP, D)
    return _mlp(groups, onehot, W1, b1, W2, b2)
